# baseline (speedup 1.0000x reference)
import functools

import jax
import jax.numpy as jnp
from jax import lax
from jax.experimental import pallas as pl
from jax.experimental.pallas import tpu as pltpu
from jax.experimental.pallas import tpu_sc as plsc

VOCAB = 1_000_000
HIDDEN = 64
B_TOTAL = 4096 * 200

_info = plsc.get_sparse_core_info()
_NC, _NS = _info.num_cores, _info.num_subcores
_NW = _NC * _NS
_B_PER_W = B_TOTAL // _NW
_CHUNK = 1024
_N_CHUNKS = _B_PER_W // _CHUNK


def _gather_body(table_hbm, idx_hbm, out_hbm, idx_v, rows_v, sem):
    wid = lax.axis_index("s") * _NC + lax.axis_index("c")
    w_base = wid * _B_PER_W

    def body(i, carry):
        base = w_base + i * _CHUNK
        pltpu.sync_copy(idx_hbm.at[pl.ds(base, _CHUNK)], idx_v)
        pltpu.async_copy(table_hbm.at[idx_v], rows_v, sem).wait()
        pltpu.sync_copy(rows_v, out_hbm.at[pl.ds(base, _CHUNK)])
        return carry

    lax.fori_loop(0, _N_CHUNKS, body, 0)


@jax.jit
def _gather(table, idx_flat):
    k = functools.partial(
        pl.kernel,
        out_type=jax.ShapeDtypeStruct((B_TOTAL, HIDDEN), jnp.float32),
        mesh=plsc.VectorSubcoreMesh(core_axis_name="c", subcore_axis_name="s"),
        scratch_types=[
            pltpu.VMEM((_CHUNK,), jnp.int32),
            pltpu.VMEM((_CHUNK, HIDDEN), jnp.float32),
            pltpu.SemaphoreType.DMA,
        ],
        compiler_params=pltpu.CompilerParams(use_tc_tiling_on_sc=False),
    )(_gather_body)
    return k(table, idx_flat)


def kernel(x, embedding_weight):
    idx_flat = x.reshape(-1).astype(jnp.int32)
    out = _gather(embedding_weight, idx_flat)
    return out.reshape(x.shape + (HIDDEN,))


# speedup vs baseline: 1.0090x; 1.0090x over previous
import functools

import jax
import jax.numpy as jnp
from jax import lax
from jax.experimental import pallas as pl
from jax.experimental.pallas import tpu as pltpu
from jax.experimental.pallas import tpu_sc as plsc

VOCAB = 1_000_000
HIDDEN = 64
B_TOTAL = 4096 * 200

_info = plsc.get_sparse_core_info()
_NC, _NS = _info.num_cores, _info.num_subcores
_NW = _NC * _NS
_B_PER_W = B_TOTAL // _NW
_NBUF = 2
_CHUNK = 800
_N_CHUNKS = _B_PER_W // _CHUNK


def _gather_body(table_hbm, idx_hbm, out_hbm, idx_v, rows_v, gsems, osems):
    wid = lax.axis_index("s") * _NC + lax.axis_index("c")
    w_base = wid * _B_PER_W

    def fetch_and_gather(c, b):
        base = w_base + c * _CHUNK
        pltpu.sync_copy(idx_hbm.at[pl.ds(base, _CHUNK)], idx_v.at[b])
        pltpu.async_copy(table_hbm.at[idx_v.at[b]], rows_v.at[b], gsems[b])

    def drain_and_store(c, b):
        base = w_base + c * _CHUNK
        pltpu.make_async_copy(
            table_hbm.at[idx_v.at[b]], rows_v.at[b], gsems[b]
        ).wait()
        pltpu.async_copy(rows_v.at[b], out_hbm.at[pl.ds(base, _CHUNK)], osems[b])

    def drain_store(c, b):
        base = w_base + c * _CHUNK
        pltpu.make_async_copy(
            rows_v.at[b], out_hbm.at[pl.ds(base, _CHUNK)], osems[b]
        ).wait()

    for b in range(_NBUF):
        fetch_and_gather(b, b)

    @pl.loop(0, _N_CHUNKS - _NBUF, step=_NBUF)
    def _(g):
        for b in range(_NBUF):
            drain_and_store(g + b, b)
            drain_store(g + b, b)
            fetch_and_gather(g + b + _NBUF, b)

    for b in range(_NBUF):
        c = _N_CHUNKS - _NBUF + b
        drain_and_store(c, b)
        drain_store(c, b)


@jax.jit
def _gather(table, idx_flat):
    k = functools.partial(
        pl.kernel,
        out_type=jax.ShapeDtypeStruct((B_TOTAL, HIDDEN), jnp.float32),
        mesh=plsc.VectorSubcoreMesh(core_axis_name="c", subcore_axis_name="s"),
        scratch_types=[
            pltpu.VMEM((_NBUF, _CHUNK), jnp.int32),
            pltpu.VMEM((_NBUF, _CHUNK, HIDDEN), jnp.float32),
            [pltpu.SemaphoreType.DMA] * _NBUF,
            [pltpu.SemaphoreType.DMA] * _NBUF,
        ],
        compiler_params=pltpu.CompilerParams(use_tc_tiling_on_sc=False),
    )(_gather_body)
    return k(table, idx_flat)


def kernel(x, embedding_weight):
    idx_flat = x.reshape(-1).astype(jnp.int32)
    out = _gather(embedding_weight, idx_flat)
    return out.reshape(x.shape + (HIDDEN,))
